# baseline (speedup 1.0000x reference)
import jax
import jax.numpy as jnp
from jax import lax
from jax.experimental import pallas as pl
from jax.experimental.pallas import tpu as pltpu
from jax.experimental.pallas import tpu_sc as plsc

ROWS = 64
N = 32768
LANES = 16
CHUNK = N // LANES
K = 4
SUB = CHUNK // K
NC, NS = 2, 16
NWORKERS = NC * NS
ROWS_PER_W = ROWS // NWORKERS
RADIX = 256
HISTK = LANES * RADIX


def _sort_body(x_hbm, out_hbm, buf_a, buf_b, h0, h1, h2, h3):
    hists = [h0, h1, h2, h3]
    lane = lax.iota(jnp.int32, LANES)
    lane_x256 = lane * RADIX
    ones = jnp.ones((LANES,), jnp.int32)
    zeros = jnp.zeros((LANES,), jnp.int32)
    msb = jnp.full((LANES,), -2147483648, jnp.int32)
    c31 = jnp.full((LANES,), 31, jnp.int32)
    m255 = jnp.full((LANES,), 255, jnp.int32)
    base_k = [lane * CHUNK + k * SUB for k in range(K)]

    def digit_of(keys_f32, shift):
        k = plsc.bitcast(keys_f32, jnp.int32)
        m = lax.shift_right_arithmetic(k, c31)
        u = lax.bitwise_xor(k, lax.bitwise_or(m, msb))
        if shift:
            u = lax.shift_right_logical(u, jnp.full((LANES,), shift, jnp.int32))
        return lax.bitwise_and(u, m255)

    wid = lax.axis_index("s") * NC + lax.axis_index("c")

    for r in range(ROWS_PER_W):
        row = wid * ROWS_PER_W + r
        pltpu.sync_copy(x_hbm.at[row], buf_a)

        for p, (src, dst) in enumerate(
            [(buf_a, buf_b), (buf_b, buf_a), (buf_a, buf_b), (buf_b, buf_a)]
        ):
            shift = 8 * p

            def zero_body(i, carry):
                for h in hists:
                    h[pl.ds(i * LANES, LANES)] = zeros
                return carry

            lax.fori_loop(0, HISTK // LANES, zero_body, 0, unroll=4)

            def hist_body(j, carry):
                for k in range(K):
                    keys = plsc.load_gather(src, [base_k[k] + j])
                    d = digit_of(keys, shift)
                    plsc.addupdate_scatter(hists[k], [lane_x256 + d], ones)
                return carry

            lax.fori_loop(0, SUB, hist_body, 0, unroll=2)

            def scan_block(b, c):
                def s1(li, tot):
                    for h in hists:
                        tot = tot + h[pl.ds(li * RADIX + b * LANES, LANES)]
                    return tot

                tot = lax.fori_loop(0, LANES, s1, zeros)
                incl = plsc.cumsum(tot)
                excl = incl - tot + c
                c_next = c + jnp.max(incl)

                def s2(li, run):
                    for h in hists:
                        sl = pl.ds(li * RADIX + b * LANES, LANES)
                        cnt = h[sl]
                        h[sl] = run
                        run = run + cnt
                    return run

                lax.fori_loop(0, LANES, s2, excl)
                return c_next

            lax.fori_loop(0, RADIX // LANES, scan_block, jnp.int32(0))

            def perm_body(j, carry):
                for k in range(K):
                    keys = plsc.load_gather(src, [base_k[k] + j])
                    d = digit_of(keys, shift)
                    addr = lane_x256 + d
                    off = plsc.load_gather(hists[k], [addr])
                    plsc.store_scatter(dst, [off], keys)
                    plsc.addupdate_scatter(hists[k], [addr], ones)
                return carry

            lax.fori_loop(0, SUB, perm_body, 0, unroll=2)

        pltpu.sync_copy(buf_a, out_hbm.at[row])


@jax.jit
def kernel(x):
    mesh = plsc.VectorSubcoreMesh(
        core_axis_name="c", subcore_axis_name="s", num_cores=NC, num_subcores=NS
    )
    run = pl.kernel(
        _sort_body,
        out_type=jax.ShapeDtypeStruct((ROWS, N), jnp.float32),
        mesh=mesh,
        scratch_types=[
            pltpu.VMEM((N,), jnp.float32),
            pltpu.VMEM((N,), jnp.float32),
            pltpu.VMEM((HISTK,), jnp.int32),
            pltpu.VMEM((HISTK,), jnp.int32),
            pltpu.VMEM((HISTK,), jnp.int32),
            pltpu.VMEM((HISTK,), jnp.int32),
        ],
        compiler_params=pltpu.CompilerParams(needs_layout_passes=False),
    )
    return run(x)


# speedup vs baseline: 1.6391x; 1.6391x over previous
import jax
import jax.numpy as jnp
from jax import lax
from jax.experimental import pallas as pl
from jax.experimental.pallas import tpu as pltpu
from jax.experimental.pallas import tpu_sc as plsc

ROWS = 64
N = 32768
LANES = 16
CHUNK = N // LANES
K = 4
SUB = CHUNK // K
NC, NS = 2, 16
NWORKERS = NC * NS
ROWS_PER_W = ROWS // NWORKERS
RADIX = 256
HISTK = RADIX * LANES
PAD = LANES


def _sort_body(x_hbm, out_hbm, buf_a, buf_b, h0, h1, h2, h3):
    hists = [h0, h1, h2, h3]
    lane = lax.iota(jnp.int32, LANES)
    ones = jnp.ones((LANES,), jnp.int32)
    zeros = jnp.zeros((LANES,), jnp.int32)
    msb = jnp.full((LANES,), -2147483648, jnp.int32)
    c31 = jnp.full((LANES,), 31, jnp.int32)
    m255 = jnp.full((LANES,), 255, jnp.int32)
    c4 = jnp.full((LANES,), 4, jnp.int32)
    csub = jnp.full((LANES,), SUB, jnp.int32)
    base_k = [lane * CHUNK + k * SUB for k in range(K)]

    def digit_of(keys_f32, shift):
        k = plsc.bitcast(keys_f32, jnp.int32)
        m = lax.shift_right_arithmetic(k, c31)
        u = lax.bitwise_xor(k, lax.bitwise_or(m, msb))
        if shift:
            u = lax.shift_right_logical(u, jnp.full((LANES,), shift, jnp.int32))
        return lax.bitwise_and(u, m255)

    wid = lax.axis_index("s") * NC + lax.axis_index("c")

    for r in range(ROWS_PER_W):
        row = wid * ROWS_PER_W + r
        pltpu.sync_copy(x_hbm.at[row], buf_a.at[pl.ds(0, N)])

        for p, (src, dst) in enumerate(
            [(buf_a, buf_b), (buf_b, buf_a), (buf_a, buf_b), (buf_b, buf_a)]
        ):
            shift = 8 * p

            def zero_body(i, carry):
                for h in hists:
                    h[pl.ds(i * LANES, LANES)] = zeros
                return carry

            lax.fori_loop(0, HISTK // LANES, zero_body, 0, unroll=4)

            def hist_body(j, carry):
                e = j - lane
                m = lax.bitwise_and(lax.ge(e, zeros), lax.lt(e, csub))
                for k in range(K):
                    keys = plsc.load_gather(src, [base_k[k] + e], mask=m)
                    d = digit_of(keys, shift)
                    addr = lax.shift_left(d, c4) + lane
                    plsc.addupdate_scatter(hists[k], [addr], ones, mask=m)
                return carry

            lax.fori_loop(0, SUB + LANES - 1, hist_body, 0, unroll=2)

            def scan_body(d, t):
                sl = pl.ds(d * LANES, LANES)
                v0, v1, v2, v3 = h0[sl], h1[sl], h2[sl], h3[sl]
                s = v0 + v1 + v2 + v3
                cs = plsc.cumsum(s)
                e0 = cs - s + t
                h0[sl] = e0
                e1 = e0 + v0
                h1[sl] = e1
                e2 = e1 + v1
                h2[sl] = e2
                h3[sl] = e2 + v2
                return t + jnp.max(cs)

            lax.fori_loop(0, RADIX, scan_body, jnp.int32(0))

            def perm_body(j, carry):
                e = j - lane
                m = lax.bitwise_and(lax.ge(e, zeros), lax.lt(e, csub))
                for k in range(K):
                    keys = plsc.load_gather(src, [base_k[k] + e], mask=m)
                    d = digit_of(keys, shift)
                    addr = lax.shift_left(d, c4) + lane
                    off = plsc.load_gather(hists[k], [addr], mask=m)
                    plsc.store_scatter(dst, [off], keys, mask=m)
                    plsc.addupdate_scatter(hists[k], [addr], ones, mask=m)
                return carry

            lax.fori_loop(0, SUB + LANES - 1, perm_body, 0, unroll=2)

        pltpu.sync_copy(buf_a.at[pl.ds(0, N)], out_hbm.at[row])


@jax.jit
def kernel(x):
    mesh = plsc.VectorSubcoreMesh(
        core_axis_name="c", subcore_axis_name="s", num_cores=NC, num_subcores=NS
    )
    run = pl.kernel(
        _sort_body,
        out_type=jax.ShapeDtypeStruct((ROWS, N), jnp.float32),
        mesh=mesh,
        scratch_types=[
            pltpu.VMEM((N + PAD,), jnp.float32),
            pltpu.VMEM((N + PAD,), jnp.float32),
            pltpu.VMEM((HISTK,), jnp.int32),
            pltpu.VMEM((HISTK,), jnp.int32),
            pltpu.VMEM((HISTK,), jnp.int32),
            pltpu.VMEM((HISTK,), jnp.int32),
        ],
        compiler_params=pltpu.CompilerParams(needs_layout_passes=False),
    )
    return run(x)


# speedup vs baseline: 4.5499x; 2.7758x over previous
import jax
import jax.numpy as jnp
from jax import lax
from jax.experimental import pallas as pl
from jax.experimental.pallas import tpu as pltpu
from jax.experimental.pallas import tpu_sc as plsc

ROWS = 64
N = 32768
LANES = 16
CHUNK = N // LANES
K = 4
SUB = CHUNK // K
NC, NS = 2, 16
NWORKERS = NC * NS
ROWS_PER_W = ROWS // NWORKERS
RADIX = 256
HISTK = RADIX * LANES
PAD = LANES


def _sort_body(x_hbm, out_hbm, buf_a, buf_b, h0, h1, h2, h3):
    hists = [h0, h1, h2, h3]
    lane = lax.iota(jnp.int32, LANES)
    ones = jnp.ones((LANES,), jnp.int32)
    zeros = jnp.zeros((LANES,), jnp.int32)
    msb = jnp.full((LANES,), -2147483648, jnp.int32)
    c31 = jnp.full((LANES,), 31, jnp.int32)
    m255 = jnp.full((LANES,), 255, jnp.int32)
    c4 = jnp.full((LANES,), 4, jnp.int32)
    csub = jnp.full((LANES,), SUB, jnp.int32)
    base_k = [lane * CHUNK + k * SUB for k in range(K)]

    def digit_of(keys_f32, shift):
        k = plsc.bitcast(keys_f32, jnp.int32)
        m = lax.shift_right_arithmetic(k, c31)
        u = lax.bitwise_xor(k, lax.bitwise_or(m, msb))
        if shift:
            u = lax.shift_right_logical(u, jnp.full((LANES,), shift, jnp.int32))
        return lax.bitwise_and(u, m255)

    wid = lax.axis_index("s") * NC + lax.axis_index("c")

    for r in range(ROWS_PER_W):
        row = wid * ROWS_PER_W + r
        pltpu.sync_copy(x_hbm.at[row], buf_a.at[pl.ds(0, N)])

        for p, (src, dst) in enumerate(
            [(buf_a, buf_b), (buf_b, buf_a), (buf_a, buf_b), (buf_b, buf_a)]
        ):
            shift = 8 * p

            @plsc.parallel_loop(0, HISTK // LANES, unroll=4)
            def _zero(i):
                for h in hists:
                    h[pl.ds(i * LANES, LANES)] = zeros

            @plsc.parallel_loop(0, SUB + LANES - 1, unroll=2)
            def _hist(j):
                e = j - lane
                m = lax.bitwise_and(lax.ge(e, zeros), lax.lt(e, csub))
                addrs = []
                for k in range(K):
                    keys = plsc.load_gather(src, [base_k[k] + e], mask=m)
                    d = digit_of(keys, shift)
                    addrs.append(lax.shift_left(d, c4) + lane)
                for k in range(K):
                    plsc.addupdate_scatter(hists[k], [addrs[k]], ones, mask=m)

            def scan_body(d, t):
                sl = pl.ds(d * LANES, LANES)
                v0, v1, v2, v3 = h0[sl], h1[sl], h2[sl], h3[sl]
                s = v0 + v1 + v2 + v3
                cs = plsc.cumsum(s)
                e0 = cs - s + t
                h0[sl] = e0
                e1 = e0 + v0
                h1[sl] = e1
                e2 = e1 + v1
                h2[sl] = e2
                h3[sl] = e2 + v2
                return t + jnp.max(cs)

            lax.fori_loop(0, RADIX, scan_body, jnp.int32(0))

            def perm_body(j, carry):
                e = j - lane
                m = lax.bitwise_and(lax.ge(e, zeros), lax.lt(e, csub))
                ks, ads = [], []
                for k in range(K):
                    keys = plsc.load_gather(src, [base_k[k] + e], mask=m)
                    d = digit_of(keys, shift)
                    ks.append(keys)
                    ads.append(lax.shift_left(d, c4) + lane)
                offs = [
                    plsc.load_gather(hists[k], [ads[k]], mask=m) for k in range(K)
                ]
                for k in range(K):
                    plsc.store_scatter(dst, [offs[k]], ks[k], mask=m)
                    plsc.addupdate_scatter(hists[k], [ads[k]], ones, mask=m)
                return carry

            lax.fori_loop(0, SUB + LANES - 1, perm_body, 0, unroll=2)

        pltpu.sync_copy(buf_a.at[pl.ds(0, N)], out_hbm.at[row])


@jax.jit
def kernel(x):
    mesh = plsc.VectorSubcoreMesh(
        core_axis_name="c", subcore_axis_name="s", num_cores=NC, num_subcores=NS
    )
    run = pl.kernel(
        _sort_body,
        out_type=jax.ShapeDtypeStruct((ROWS, N), jnp.float32),
        mesh=mesh,
        scratch_types=[
            pltpu.VMEM((N + PAD,), jnp.float32),
            pltpu.VMEM((N + PAD,), jnp.float32),
            pltpu.VMEM((HISTK,), jnp.int32),
            pltpu.VMEM((HISTK,), jnp.int32),
            pltpu.VMEM((HISTK,), jnp.int32),
            pltpu.VMEM((HISTK,), jnp.int32),
        ],
        compiler_params=pltpu.CompilerParams(needs_layout_passes=False),
    )
    return run(x)


# speedup vs baseline: 6.9279x; 1.5226x over previous
import jax
import jax.numpy as jnp
from jax import lax
from jax.experimental import pallas as pl
from jax.experimental.pallas import tpu as pltpu
from jax.experimental.pallas import tpu_sc as plsc

ROWS = 64
N = 32768
LANES = 16
CHUNK = N // LANES
K = 4
SUB = CHUNK // K
NC, NS = 2, 16
NWORKERS = NC * NS
ROWS_PER_W = ROWS // NWORKERS
RADIX = 256
HISTK = RADIX * LANES
PAD = LANES


def _sort_body(x_hbm, out_hbm, buf_a, buf_b, h0, h1, h2, h3):
    hists = [h0, h1, h2, h3]
    lane = lax.iota(jnp.int32, LANES)
    ones = jnp.ones((LANES,), jnp.int32)
    zeros = jnp.zeros((LANES,), jnp.int32)
    msb = jnp.full((LANES,), -2147483648, jnp.int32)
    c31 = jnp.full((LANES,), 31, jnp.int32)
    m255 = jnp.full((LANES,), 255, jnp.int32)
    c4 = jnp.full((LANES,), 4, jnp.int32)
    csub = jnp.full((LANES,), SUB, jnp.int32)
    base_k = [lane * CHUNK + k * SUB for k in range(K)]

    def digit_of(keys_f32, shift):
        k = plsc.bitcast(keys_f32, jnp.int32)
        m = lax.shift_right_arithmetic(k, c31)
        u = lax.bitwise_xor(k, lax.bitwise_or(m, msb))
        if shift:
            u = lax.shift_right_logical(u, jnp.full((LANES,), shift, jnp.int32))
        return lax.bitwise_and(u, m255)

    wid = lax.axis_index("s") * NC + lax.axis_index("c")

    for r in range(ROWS_PER_W):
        row = wid * ROWS_PER_W + r
        pltpu.sync_copy(x_hbm.at[row], buf_a.at[pl.ds(0, N)])

        for p, (src, dst) in enumerate(
            [(buf_a, buf_b), (buf_b, buf_a), (buf_a, buf_b), (buf_b, buf_a)]
        ):
            shift = 8 * p

            @plsc.parallel_loop(0, HISTK // LANES, unroll=4)
            def _zero(i):
                for h in hists:
                    h[pl.ds(i * LANES, LANES)] = zeros

            @plsc.parallel_loop(0, SUB + LANES - 1, unroll=2)
            def _hist(j):
                e = j - lane
                m = lax.bitwise_and(lax.ge(e, zeros), lax.lt(e, csub))
                addrs = []
                for k in range(K):
                    keys = plsc.load_gather(src, [base_k[k] + e], mask=m)
                    d = digit_of(keys, shift)
                    addrs.append(lax.shift_left(d, c4) + lane)
                for k in range(K):
                    plsc.addupdate_scatter(hists[k], [addrs[k]], ones, mask=m)

            @plsc.parallel_loop(0, RADIX, unroll=2, carry=jnp.int32(0))
            def _scan(d, t):
                sl = pl.ds(d * LANES, LANES)
                v0, v1, v2, v3 = h0[sl], h1[sl], h2[sl], h3[sl]
                s = v0 + v1 + v2 + v3
                cs = plsc.cumsum(s)
                e0 = cs - s + t
                h0[sl] = e0
                e1 = e0 + v0
                h1[sl] = e1
                e2 = e1 + v1
                h2[sl] = e2
                h3[sl] = e2 + v2
                return t + jnp.max(cs)

            def perm_fetch(j):
                e = j - lane
                m = lax.bitwise_and(lax.ge(e, zeros), lax.lt(e, csub))
                ks, ads = [], []
                for k in range(K):
                    keys = plsc.load_gather(src, [base_k[k] + e], mask=m)
                    d = digit_of(keys, shift)
                    ks.append(keys)
                    ads.append(lax.shift_left(d, c4) + lane)
                return tuple(ks), tuple(ads), m

            def perm_body(j, st):
                ks, ads, m = st
                nxt = perm_fetch(j + 1)
                offs = [
                    plsc.load_gather(hists[k], [ads[k]], mask=m) for k in range(K)
                ]
                for k in range(K):
                    plsc.store_scatter(dst, [offs[k]], ks[k], mask=m)
                    plsc.addupdate_scatter(hists[k], [ads[k]], ones, mask=m)
                return nxt

            lax.fori_loop(
                0, SUB + LANES - 1, perm_body, perm_fetch(jnp.int32(0)), unroll=2
            )

        pltpu.sync_copy(buf_a.at[pl.ds(0, N)], out_hbm.at[row])


@jax.jit
def kernel(x):
    mesh = plsc.VectorSubcoreMesh(
        core_axis_name="c", subcore_axis_name="s", num_cores=NC, num_subcores=NS
    )
    run = pl.kernel(
        _sort_body,
        out_type=jax.ShapeDtypeStruct((ROWS, N), jnp.float32),
        mesh=mesh,
        scratch_types=[
            pltpu.VMEM((N + PAD,), jnp.float32),
            pltpu.VMEM((N + PAD,), jnp.float32),
            pltpu.VMEM((HISTK,), jnp.int32),
            pltpu.VMEM((HISTK,), jnp.int32),
            pltpu.VMEM((HISTK,), jnp.int32),
            pltpu.VMEM((HISTK,), jnp.int32),
        ],
        compiler_params=pltpu.CompilerParams(needs_layout_passes=False),
    )
    return run(x)


# speedup vs baseline: 7.2354x; 1.0444x over previous
import jax
import jax.numpy as jnp
from jax import lax
from jax.experimental import pallas as pl
from jax.experimental.pallas import tpu as pltpu
from jax.experimental.pallas import tpu_sc as plsc

ROWS = 64
N = 32768
LANES = 16
CHUNK = N // LANES
K = 4
SUB = CHUNK // K
NC, NS = 2, 16
NWORKERS = NC * NS
ROWS_PER_W = ROWS // NWORKERS
RADIX = 256
HISTK = RADIX * LANES
PAD = LANES


def _sort_body(x_hbm, out_hbm, buf_a, buf_b, buf_c, h0, h1, h2, h3, sem_in, sem_out):
    hists = [h0, h1, h2, h3]
    lane = lax.iota(jnp.int32, LANES)
    ones = jnp.ones((LANES,), jnp.int32)
    zeros = jnp.zeros((LANES,), jnp.int32)
    msb = jnp.full((LANES,), -2147483648, jnp.int32)
    c31 = jnp.full((LANES,), 31, jnp.int32)
    m255 = jnp.full((LANES,), 255, jnp.int32)
    c4 = jnp.full((LANES,), 4, jnp.int32)
    csub = jnp.full((LANES,), SUB, jnp.int32)
    base_k = [lane * CHUNK + k * SUB for k in range(K)]

    def digit_of(keys_f32, shift):
        k = plsc.bitcast(keys_f32, jnp.int32)
        m = lax.shift_right_arithmetic(k, c31)
        u = lax.bitwise_xor(k, lax.bitwise_or(m, msb))
        if shift:
            u = lax.shift_right_logical(u, jnp.full((LANES,), shift, jnp.int32))
        return lax.bitwise_and(u, m255)

    wid = lax.axis_index("s") * NC + lax.axis_index("c")

    def sort_row(buf_x, buf_y):
        for p, (src, dst) in enumerate(
            [(buf_x, buf_y), (buf_y, buf_x), (buf_x, buf_y), (buf_y, buf_x)]
        ):
            shift = 8 * p

            @plsc.parallel_loop(0, HISTK // LANES, unroll=4)
            def _zero(i):
                for h in hists:
                    h[pl.ds(i * LANES, LANES)] = zeros

            def hist_fetch(j):
                e = j - lane
                m = lax.bitwise_and(lax.ge(e, zeros), lax.lt(e, csub))
                addrs = []
                for k in range(K):
                    keys = plsc.load_gather(src, [base_k[k] + e], mask=m)
                    d = digit_of(keys, shift)
                    addrs.append(lax.shift_left(d, c4) + lane)
                return tuple(addrs), m

            @plsc.parallel_loop(
                0, SUB + LANES - 1, unroll=4, carry=hist_fetch(jnp.int32(0))
            )
            def _hist(j, st):
                addrs, m = st
                nxt = hist_fetch(j + 1)
                for k in range(K):
                    plsc.addupdate_scatter(hists[k], [addrs[k]], ones, mask=m)
                return nxt

            @plsc.parallel_loop(0, RADIX, unroll=2, carry=jnp.int32(0))
            def _scan(d, t):
                sl = pl.ds(d * LANES, LANES)
                v0, v1, v2, v3 = h0[sl], h1[sl], h2[sl], h3[sl]
                s = v0 + v1 + v2 + v3
                cs = plsc.cumsum(s)
                e0 = cs - s + t
                h0[sl] = e0
                e1 = e0 + v0
                h1[sl] = e1
                e2 = e1 + v1
                h2[sl] = e2
                h3[sl] = e2 + v2
                return t + jnp.max(cs)

            def perm_fetch(j):
                e = j - lane
                m = lax.bitwise_and(lax.ge(e, zeros), lax.lt(e, csub))
                ks, ads = [], []
                for k in range(K):
                    keys = plsc.load_gather(src, [base_k[k] + e], mask=m)
                    d = digit_of(keys, shift)
                    ks.append(keys)
                    ads.append(lax.shift_left(d, c4) + lane)
                return tuple(ks), tuple(ads), m

            def perm_body(j, st):
                ks, ads, m = st
                nxt = perm_fetch(j + 1)
                offs = [
                    plsc.load_gather(hists[k], [ads[k]], mask=m) for k in range(K)
                ]
                for k in range(K):
                    plsc.store_scatter(dst, [offs[k]], ks[k], mask=m)
                    plsc.addupdate_scatter(hists[k], [ads[k]], ones, mask=m)
                return nxt

            lax.fori_loop(
                0, SUB + LANES - 1, perm_body, perm_fetch(jnp.int32(0)), unroll=2
            )

    row0 = wid * ROWS_PER_W
    row1 = row0 + 1
    pltpu.sync_copy(x_hbm.at[row0], buf_a.at[pl.ds(0, N)])
    cp_in = pltpu.async_copy(x_hbm.at[row1], buf_c.at[pl.ds(0, N)], sem_in)
    sort_row(buf_a, buf_b)
    cp_out = pltpu.async_copy(buf_a.at[pl.ds(0, N)], out_hbm.at[row0], sem_out)
    cp_in.wait()
    sort_row(buf_c, buf_b)
    cp_out.wait()
    pltpu.sync_copy(buf_c.at[pl.ds(0, N)], out_hbm.at[row1])


@jax.jit
def kernel(x):
    mesh = plsc.VectorSubcoreMesh(
        core_axis_name="c", subcore_axis_name="s", num_cores=NC, num_subcores=NS
    )
    run = pl.kernel(
        _sort_body,
        out_type=jax.ShapeDtypeStruct((ROWS, N), jnp.float32),
        mesh=mesh,
        scratch_types=[
            pltpu.VMEM((N + PAD,), jnp.float32),
            pltpu.VMEM((N + PAD,), jnp.float32),
            pltpu.VMEM((N + PAD,), jnp.float32),
            pltpu.VMEM((HISTK,), jnp.int32),
            pltpu.VMEM((HISTK,), jnp.int32),
            pltpu.VMEM((HISTK,), jnp.int32),
            pltpu.VMEM((HISTK,), jnp.int32),
            pltpu.SemaphoreType.DMA,
            pltpu.SemaphoreType.DMA,
        ],
        compiler_params=pltpu.CompilerParams(needs_layout_passes=False),
    )
    return run(x)
